# baseline (speedup 1.0000x reference)
import jax
import jax.numpy as jnp
from jax.experimental import pallas as pl
from jax.experimental.pallas import tpu as pltpu

_TS = 256
_TF = 1024


def _body(x_ref, o_ref, carry_ref):
    s = pl.program_id(2)

    @pl.when(s == 0)
    def _():
        carry_ref[...] = jnp.zeros_like(carry_ref)

    xb = x_ref[0]
    r = jax.lax.broadcasted_iota(jnp.int32, (_TS, _TS), 0)
    cc = jax.lax.broadcasted_iota(jnp.int32, (_TS, _TS), 1)
    tril = (r >= cc).astype(jnp.float32)
    c = jax.lax.dot(tril, xb, preferred_element_type=jnp.float32)
    c = c + carry_ref[...]
    o_ref[0] = c
    carry_ref[...] = c[_TS - 1:_TS, :]


def kernel(x, dim):
    B, S, F = x.shape
    grid = (B, F // _TF, S // _TS)
    return pl.pallas_call(
        _body,
        grid=grid,
        in_specs=[pl.BlockSpec((1, _TS, _TF), lambda b, f, s: (b, s, f))],
        out_specs=pl.BlockSpec((1, _TS, _TF), lambda b, f, s: (b, s, f)),
        out_shape=jax.ShapeDtypeStruct((B, S, F), x.dtype),
        scratch_shapes=[pltpu.VMEM((1, _TF), jnp.float32)],
    )(x)


# speedup vs baseline: 1.5923x; 1.5923x over previous
import jax
import jax.numpy as jnp
from jax.experimental import pallas as pl
from jax.experimental.pallas import tpu as pltpu

_TS = 512
_TF = 2048


def _body(x_ref, o_ref, carry_ref):
    s = pl.program_id(2)

    @pl.when(s == 0)
    def _():
        carry_ref[...] = jnp.zeros_like(carry_ref)

    xb = x_ref[0]
    r = jax.lax.broadcasted_iota(jnp.int32, (_TS, _TS), 0)
    cc = jax.lax.broadcasted_iota(jnp.int32, (_TS, _TS), 1)
    tril = (r >= cc).astype(jnp.float32)
    c = jax.lax.dot(tril, xb, preferred_element_type=jnp.float32)
    c = c + carry_ref[...]
    o_ref[0] = c
    carry_ref[...] = c[_TS - 1:_TS, :]


def kernel(x, dim):
    B, S, F = x.shape
    grid = (B, F // _TF, S // _TS)
    return pl.pallas_call(
        _body,
        grid=grid,
        in_specs=[pl.BlockSpec((1, _TS, _TF), lambda b, f, s: (b, s, f))],
        out_specs=pl.BlockSpec((1, _TS, _TF), lambda b, f, s: (b, s, f)),
        out_shape=jax.ShapeDtypeStruct((B, S, F), x.dtype),
        scratch_shapes=[pltpu.VMEM((1, _TF), jnp.float32)],
    )(x)
